# baseline (speedup 1.0000x reference)
import functools

import jax
import jax.numpy as jnp
from jax import lax
from jax.experimental import pallas as pl
from jax.experimental.pallas import tpu as pltpu
from jax.experimental.pallas import tpu_sc as plsc

_N = 10000
_E = 320000
_D = 128
_CH = 128
_NC = 2
_NS = 16
_NW = _NC * _NS
_STRIPE = 624
_LAST_STRIPE = _N - 15 * _STRIPE
_CPW = 80
_NCHUNK = _CPW * _NW
_EPAD = _NCHUNK * _CH
_PHASES = 5
_PCH = _CPW // _PHASES


def _make_side_kernel(real_d):
  mesh = plsc.VectorSubcoreMesh(core_axis_name="c", subcore_axis_name="s",
                                num_cores=_NC, num_subcores=_NS)

  def body(ego, srcm, dstm, wm, zeros_hbm, out, acc, src_v, dst_v, w_v,
           rows0, rows1, sem0, sem1, ssem0, ssem1):
    cid = lax.axis_index("c")
    sid = lax.axis_index("s")
    wid = sid * _NC + cid
    r0 = pl.multiple_of(sid * _STRIPE, 8)

    @pl.when(sid < _NS - 1)
    def _():
      pltpu.sync_copy(zeros_hbm.at[pl.ds(r0, _STRIPE)],
                      acc.at[pl.ds(r0, _STRIPE)])

    @pl.when(sid == _NS - 1)
    def _():
      pltpu.sync_copy(zeros_hbm.at[pl.ds(15 * _STRIPE, _LAST_STRIPE)],
                      acc.at[pl.ds(15 * _STRIPE, _LAST_STRIPE)])

    plsc.subcore_barrier()

    def scale_half(rows, j, h):
      def scale_body(t, c2):
        w16 = w_v[j, pl.ds(t * 16, 16)]
        for l in range(16):
          ws = w16[l]
          e = t * 16 + l
          for k in range(real_d // 16):
            sl = pl.ds(k * 16, 16)
            rows[e, sl] = rows[e, sl] * ws
        return c2

      lax.fori_loop(4 * h, 4 * h + 4, scale_body, 0)

    def half_scatters(rows, j, ssem):
      scale_half(rows, j, 0)
      s0 = pltpu.async_copy(rows.at[pl.ds(0, _CH // 2)],
                            acc.at[dst_v.at[2 * j]], ssem, add=True)
      scale_half(rows, j, 1)
      s1 = pltpu.async_copy(rows.at[pl.ds(_CH // 2, _CH // 2)],
                            acc.at[dst_v.at[2 * j + 1]], ssem, add=True)
      return s0, s1

    def phase_body(p, carry):
      c0 = pl.multiple_of(wid * _CPW + p * _PCH, 8)
      pltpu.sync_copy(srcm.at[pl.ds(c0, _PCH)], src_v)
      pltpu.sync_copy(dstm.at[pl.ds(2 * c0, 2 * _PCH)], dst_v)
      pltpu.sync_copy(wm.at[pl.ds(c0, _PCH)], w_v)

      pltpu.async_copy(ego.at[src_v.at[0]], rows0, sem0)
      pltpu.async_copy(ego.at[src_v.at[1]], rows1, sem1)

      def chunk_body(jj, c2):
        a = 2 * jj
        b = a + 1
        pltpu.make_async_copy(ego.at[src_v.at[0]], rows0, sem0).wait()
        sa0, sa1 = half_scatters(rows0, a, ssem0)

        pltpu.make_async_copy(ego.at[src_v.at[1]], rows1, sem1).wait()
        sb0, sb1 = half_scatters(rows1, b, ssem1)

        sa0.wait()
        sa1.wait()

        @pl.when(a + 2 < _PCH)
        def _():
          pltpu.async_copy(ego.at[src_v.at[a + 2]], rows0, sem0)

        sb0.wait()
        sb1.wait()

        @pl.when(b + 2 < _PCH)
        def _():
          pltpu.async_copy(ego.at[src_v.at[b + 2]], rows1, sem1)

        return c2

      lax.fori_loop(0, _PCH // 2, chunk_body, 0)
      return carry

    lax.fori_loop(0, _PHASES, phase_body, 0)
    plsc.subcore_barrier()

    @pl.when(sid < _NS - 1)
    def _():
      pltpu.sync_copy(acc.at[pl.ds(r0, _STRIPE)],
                      out.at[cid, pl.ds(r0, _STRIPE)])

    @pl.when(sid == _NS - 1)
    def _():
      pltpu.sync_copy(acc.at[pl.ds(15 * _STRIPE, _LAST_STRIPE)],
                      out.at[cid, pl.ds(15 * _STRIPE, _LAST_STRIPE)])

  return pl.kernel(
      body,
      out_type=jax.ShapeDtypeStruct((_NC, _N, _D), jnp.float32),
      mesh=mesh,
      scratch_types=[
          pltpu.VMEM_SHARED((_N, _D), jnp.float32),
          pltpu.VMEM((_PCH, _CH), jnp.int32),
          pltpu.VMEM((2 * _PCH, _CH // 2), jnp.int32),
          pltpu.VMEM((_PCH, _CH), jnp.float32),
          pltpu.VMEM((_CH, _D), jnp.float32),
          pltpu.VMEM((_CH, _D), jnp.float32),
          pltpu.SemaphoreType.DMA,
          pltpu.SemaphoreType.DMA,
          pltpu.SemaphoreType.DMA,
          pltpu.SemaphoreType.DMA,
      ],
  )


def _make_dense_kernel(Do, blk):
  def body(ego_ref, p0_ref, p1_ref, w1_ref, b1_ref, w2_ref, b2_ref,
           eg_ref, nm_ref):
    ego = ego_ref[...]
    side = p0_ref[...] + p1_ref[...]
    h1 = jnp.dot(ego + side, w1_ref[...],
                 preferred_element_type=jnp.float32) + b1_ref[...]
    h1 = jnp.where(h1 >= 0, h1, 0.01 * h1)
    h2 = jnp.dot(ego * side, w2_ref[...],
                 preferred_element_type=jnp.float32) + b2_ref[...]
    h2 = jnp.where(h2 >= 0, h2, 0.01 * h2)
    eg = h1 + h2
    eg_ref[...] = jnp.concatenate(
        [eg, jnp.zeros((eg.shape[0], _D - Do), jnp.float32)], axis=1)
    nrm = jnp.sqrt(jnp.sum(eg * eg, axis=1, keepdims=True))
    nm_ref[...] = eg / jnp.maximum(nrm, 1e-12)

  return pl.pallas_call(
      body,
      grid=(_N // blk,),
      in_specs=[
          pl.BlockSpec((blk, _D), lambda i: (i, 0)),
          pl.BlockSpec((blk, _D), lambda i: (i, 0)),
          pl.BlockSpec((blk, _D), lambda i: (i, 0)),
          pl.BlockSpec((_D, Do), lambda i: (0, 0)),
          pl.BlockSpec((1, Do), lambda i: (0, 0)),
          pl.BlockSpec((_D, Do), lambda i: (0, 0)),
          pl.BlockSpec((1, Do), lambda i: (0, 0)),
      ],
      out_specs=[
          pl.BlockSpec((blk, _D), lambda i: (i, 0)),
          pl.BlockSpec((blk, Do), lambda i: (i, 0)),
      ],
      out_shape=[
          jax.ShapeDtypeStruct((_N, _D), jnp.float32),
          jax.ShapeDtypeStruct((_N, Do), jnp.float32),
      ],
  )


_DIMS = [(128, 64), (64, 32), (32, 16)]
_SIDE = {D: _make_side_kernel(D) for D, _ in _DIMS}
_DENSE = {Do: _make_dense_kernel(Do, 2000) for _, Do in _DIMS}


def kernel(x, edge_index, edge_weight, W1_0, b1_0, W2_0, b2_0, W1_1, b1_1,
           W2_1, b2_1, W1_2, b1_2, W2_2, b2_2):
  pad = _EPAD - _E
  pad_idx = (jnp.arange(pad, dtype=jnp.int32) * 37) % _N
  src = jnp.concatenate([edge_index[0], pad_idx]).reshape(_NCHUNK, _CH)
  dst = jnp.concatenate([edge_index[1], pad_idx]).reshape(2 * _NCHUNK,
                                                          _CH // 2)
  wm = jnp.pad(edge_weight, (0, pad)).reshape(_NCHUNK, _CH)
  params = [(W1_0, b1_0, W2_0, b2_0), (W1_1, b1_1, W2_1, b2_1),
            (W1_2, b1_2, W2_2, b2_2)]
  zeros = jnp.zeros((_N, _D), jnp.float32)
  ego = x
  outs = [x]
  for (W1, b1, W2, b2), (D, Do) in zip(params, _DIMS):
    W1p = jnp.pad(W1, ((0, _D - D), (0, 0)))
    W2p = jnp.pad(W2, ((0, _D - D), (0, 0)))
    parts = _SIDE[D](ego, src, dst, wm, zeros)
    eg, nm = _DENSE[Do](ego, parts[0], parts[1], W1p, b1.reshape(1, Do),
                        W2p, b2.reshape(1, Do))
    ego = eg
    outs.append(nm)
  return jnp.concatenate(outs, axis=1)


# speedup vs baseline: 1.1914x; 1.1914x over previous
import functools

import jax
import jax.numpy as jnp
from jax import lax
from jax.experimental import pallas as pl
from jax.experimental.pallas import tpu as pltpu
from jax.experimental.pallas import tpu_sc as plsc

_N = 10000
_E = 320000
_D = 128
_CH = 128
_NC = 2
_NS = 16
_NW = _NC * _NS
_STRIPE = 624
_LAST_STRIPE = _N - 15 * _STRIPE
_CPW = 80
_NCHUNK = _CPW * _NW
_EPAD = _NCHUNK * _CH
_PHASES = 2
_PCH = _CPW // _PHASES


def _make_side_kernel(real_d):
  mesh = plsc.VectorSubcoreMesh(core_axis_name="c", subcore_axis_name="s",
                                num_cores=_NC, num_subcores=_NS)

  def body(ego, srcm, dstm, wm, zeros_hbm, out, acc, src_v, dst_v, w_v,
           rows0, rows1, sem0, sem1):
    cid = lax.axis_index("c")
    sid = lax.axis_index("s")
    wid = sid * _NC + cid
    r0 = pl.multiple_of(sid * _STRIPE, 8)

    @pl.when(sid < _NS - 1)
    def _():
      pltpu.sync_copy(zeros_hbm.at[pl.ds(r0, _STRIPE)],
                      acc.at[pl.ds(r0, _STRIPE)])

    @pl.when(sid == _NS - 1)
    def _():
      pltpu.sync_copy(zeros_hbm.at[pl.ds(15 * _STRIPE, _LAST_STRIPE)],
                      acc.at[pl.ds(15 * _STRIPE, _LAST_STRIPE)])

    plsc.subcore_barrier()

    def scale(rows, j):
      def scale_body(t, c2):
        w16 = w_v[j, pl.ds(t * 16, 16)]
        for l in range(16):
          ws = w16[l]
          e = t * 16 + l
          for k in range(real_d // 16):
            sl = pl.ds(k * 16, 16)
            rows[e, sl] = rows[e, sl] * ws
        return c2

      lax.fori_loop(0, _CH // 16, scale_body, 0)

    bufs = (rows0, rows1)
    sems = (sem0, sem1)

    def phase_body(p, carry):
      c0 = pl.multiple_of(wid * _CPW + p * _PCH, 8)
      pltpu.sync_copy(srcm.at[pl.ds(c0, _PCH)], src_v)
      pltpu.sync_copy(dstm.at[pl.ds(c0, _PCH)], dst_v)
      pltpu.sync_copy(wm.at[pl.ds(c0, _PCH)], w_v)

      pltpu.async_copy(ego.at[src_v.at[0]], rows0, sem0)
      pltpu.async_copy(ego.at[src_v.at[1]], rows1, sem1)

      def chunk_body(jj, c2):
        a = 2 * jj
        b = a + 1
        pltpu.make_async_copy(ego.at[src_v.at[0]], rows0, sem0).wait()
        scale(rows0, a)
        pltpu.sync_copy(rows0, acc.at[dst_v.at[a]], add=True)

        @pl.when(a + 2 < _PCH)
        def _():
          pltpu.async_copy(ego.at[src_v.at[a + 2]], rows0, sem0)

        pltpu.make_async_copy(ego.at[src_v.at[1]], rows1, sem1).wait()
        scale(rows1, b)
        pltpu.sync_copy(rows1, acc.at[dst_v.at[b]], add=True)

        @pl.when(b + 2 < _PCH)
        def _():
          pltpu.async_copy(ego.at[src_v.at[b + 2]], rows1, sem1)

        return c2

      lax.fori_loop(0, _PCH // 2, chunk_body, 0)
      return carry

    lax.fori_loop(0, _PHASES, phase_body, 0)
    plsc.subcore_barrier()

    @pl.when(sid < _NS - 1)
    def _():
      pltpu.sync_copy(acc.at[pl.ds(r0, _STRIPE)],
                      out.at[cid, pl.ds(r0, _STRIPE)])

    @pl.when(sid == _NS - 1)
    def _():
      pltpu.sync_copy(acc.at[pl.ds(15 * _STRIPE, _LAST_STRIPE)],
                      out.at[cid, pl.ds(15 * _STRIPE, _LAST_STRIPE)])

  return pl.kernel(
      body,
      out_type=jax.ShapeDtypeStruct((_NC, _N, _D), jnp.float32),
      mesh=mesh,
      scratch_types=[
          pltpu.VMEM_SHARED((_N, _D), jnp.float32),
          pltpu.VMEM((_PCH, _CH), jnp.int32),
          pltpu.VMEM((_PCH, _CH), jnp.int32),
          pltpu.VMEM((_PCH, _CH), jnp.float32),
          pltpu.VMEM((_CH, _D), jnp.float32),
          pltpu.VMEM((_CH, _D), jnp.float32),
          pltpu.SemaphoreType.DMA,
          pltpu.SemaphoreType.DMA,
      ],
  )


def _make_dense_kernel(Do, blk):
  def body(ego_ref, p0_ref, p1_ref, w1_ref, b1_ref, w2_ref, b2_ref,
           eg_ref, nm_ref):
    ego = ego_ref[...]
    side = p0_ref[...] + p1_ref[...]
    h1 = jnp.dot(ego + side, w1_ref[...],
                 preferred_element_type=jnp.float32) + b1_ref[...]
    h1 = jnp.where(h1 >= 0, h1, 0.01 * h1)
    h2 = jnp.dot(ego * side, w2_ref[...],
                 preferred_element_type=jnp.float32) + b2_ref[...]
    h2 = jnp.where(h2 >= 0, h2, 0.01 * h2)
    eg = h1 + h2
    eg_ref[...] = jnp.concatenate(
        [eg, jnp.zeros((eg.shape[0], _D - Do), jnp.float32)], axis=1)
    nrm = jnp.sqrt(jnp.sum(eg * eg, axis=1, keepdims=True))
    nm_ref[...] = eg / jnp.maximum(nrm, 1e-12)

  return pl.pallas_call(
      body,
      grid=(_N // blk,),
      in_specs=[
          pl.BlockSpec((blk, _D), lambda i: (i, 0)),
          pl.BlockSpec((blk, _D), lambda i: (i, 0)),
          pl.BlockSpec((blk, _D), lambda i: (i, 0)),
          pl.BlockSpec((_D, Do), lambda i: (0, 0)),
          pl.BlockSpec((1, Do), lambda i: (0, 0)),
          pl.BlockSpec((_D, Do), lambda i: (0, 0)),
          pl.BlockSpec((1, Do), lambda i: (0, 0)),
      ],
      out_specs=[
          pl.BlockSpec((blk, _D), lambda i: (i, 0)),
          pl.BlockSpec((blk, Do), lambda i: (i, 0)),
      ],
      out_shape=[
          jax.ShapeDtypeStruct((_N, _D), jnp.float32),
          jax.ShapeDtypeStruct((_N, Do), jnp.float32),
      ],
  )


_DIMS = [(128, 64), (64, 32), (32, 16)]
_SIDE = {D: _make_side_kernel(D) for D, _ in _DIMS}
_DENSE = {Do: _make_dense_kernel(Do, 2000) for _, Do in _DIMS}


def kernel(x, edge_index, edge_weight, W1_0, b1_0, W2_0, b2_0, W1_1, b1_1,
           W2_1, b2_1, W1_2, b1_2, W2_2, b2_2):
  pad = _EPAD - _E
  pad_idx = (jnp.arange(pad, dtype=jnp.int32) * 37) % _N
  src = jnp.concatenate([edge_index[0], pad_idx]).reshape(_NCHUNK, _CH)
  dst = jnp.concatenate([edge_index[1], pad_idx]).reshape(_NCHUNK, _CH)
  wm = jnp.pad(edge_weight, (0, pad)).reshape(_NCHUNK, _CH)
  params = [(W1_0, b1_0, W2_0, b2_0), (W1_1, b1_1, W2_1, b2_1),
            (W1_2, b1_2, W2_2, b2_2)]
  zeros = jnp.zeros((_N, _D), jnp.float32)
  ego = x
  outs = [x]
  for (W1, b1, W2, b2), (D, Do) in zip(params, _DIMS):
    W1p = jnp.pad(W1, ((0, _D - D), (0, 0)))
    W2p = jnp.pad(W2, ((0, _D - D), (0, 0)))
    parts = _SIDE[D](ego, src, dst, wm, zeros)
    eg, nm = _DENSE[Do](ego, parts[0], parts[1], W1p, b1.reshape(1, Do),
                        W2p, b2.reshape(1, Do))
    ego = eg
    outs.append(nm)
  return jnp.concatenate(outs, axis=1)
